# baseline (speedup 1.0000x reference)
import functools

import jax
import jax.numpy as jnp
from jax import lax
from jax.experimental import pallas as pl
from jax.experimental.pallas import tpu as pltpu
from jax.experimental.pallas import tpu_sc as plsc

B = 4096
HIST = 50
D = 64
VOCAB1 = 1000001
NC = 2
NS = 16
NW = NC * NS
PW = B // NW
IPS = 2 * HIST
CS = 4
CHUNK_ROWS = CS * IPS
N_CH = PW // CS
L = 16

IDQ = 1 << 18
ID_ROWS = IDQ
VI_COLS = 1000000
RB = 4096


def _id_repack_body(q0, q1, q2, q3, _textp, out):
    out[...] = jnp.concatenate(
        [jnp.transpose(q0[...]), jnp.transpose(q1[...]),
         jnp.transpose(q2[...]), jnp.transpose(q3[...])], axis=1)


def _repack_id(tabT, textp):
    nblk = ID_ROWS // RB
    vblk = (VI_COLS + RB - 1) // RB - 1
    return pl.pallas_call(
        _id_repack_body,
        grid=(nblk,),
        in_specs=[
            pl.BlockSpec((32, RB),
                         lambda i, _q=q, _n=nblk, _v=vblk:
                         (0, jnp.minimum(_q * _n + i, _v)))
            for q in range(4)
        ] + [pl.BlockSpec((8, 128), lambda i: (0, 0))],
        out_specs=pl.BlockSpec((RB, 128), lambda i: (i, 0)),
        out_shape=jax.ShapeDtypeStruct((ID_ROWS, 128), jnp.float32),
        compiler_params=pltpu.CompilerParams(
            dimension_semantics=("parallel",)),
    )(tabT, tabT, tabT, tabT, textp)


@functools.cache
def _make_sc_pool():
    mesh = plsc.VectorSubcoreMesh(core_axis_name="c", subcore_axis_name="s",
                                  num_cores=NC, num_subcores=NS)
    return functools.partial(
        pl.kernel,
        out_type=jax.ShapeDtypeStruct((B, 128), jnp.float32),
        mesh=mesh,
        scratch_types=[
            pltpu.VMEM((PW * IPS,), jnp.int32),
            pltpu.VMEM((CHUNK_ROWS, 128), jnp.float32),
            pltpu.VMEM((PW, 128), jnp.float32),
            pltpu.SemaphoreType.DMA,
        ],
        compiler_params=pltpu.CompilerParams(use_tc_tiling_on_sc=True),
    )(_sc_pool_body)


@functools.cache
def _make_sc_id_gather():
    mesh = plsc.VectorSubcoreMesh(core_axis_name="c", subcore_axis_name="s",
                                  num_cores=NC, num_subcores=NS)
    return functools.partial(
        pl.kernel,
        out_type=(
            jax.ShapeDtypeStruct((B, 128), jnp.float32),
            jax.ShapeDtypeStruct((B, 128), jnp.float32),
        ),
        mesh=mesh,
        scratch_types=[
            pltpu.VMEM((PW,), jnp.int32),
            pltpu.VMEM((PW, 128), jnp.float32),
            pltpu.SemaphoreType.DMA,
        ],
        compiler_params=pltpu.CompilerParams(use_tc_tiling_on_sc=True),
    )(_sc_id_gather_body)


def _sc_pool_body(text_hbm, hidx_hbm, pooled_hbm, idx_v, rows_v, out_v, sem):
    wid = lax.axis_index("s") * NC + lax.axis_index("c")
    base = wid * PW * IPS

    pltpu.sync_copy(hidx_hbm.at[pl.ds(base, PW * IPS)], idx_v)

    @pl.loop(0, N_CH)
    def _chunk(ch):
        row0 = ch * CHUNK_ROWS
        cps = []
        for j, w in ((0, 128), (1, 128), (2, 128), (3, 16)):
            cps.append(pltpu.async_copy(
                text_hbm.at[idx_v.at[pl.ds(row0 + j * 128, w)]],
                rows_v.at[pl.ds(j * 128, w)], sem))
        for cp in cps:
            cp.wait()
        for s in range(CS):
            sp = ch * CS + s
            accs = [jnp.zeros((L,), jnp.float32) for _ in range(8)]
            for half, rbase in ((0, s * IPS), (1, s * IPS + HIST)):
                for r in range(HIST):
                    for c in range(4):
                        accs[4 * half + c] = accs[4 * half + c] + \
                            rows_v[rbase + r, pl.ds(c * L, L)]
            for c in range(8):
                out_v[sp, pl.ds(c * L, L)] = accs[c] * jnp.float32(1.0 / HIST)

    pltpu.sync_copy(out_v, pooled_hbm.at[pl.ds(wid * PW, PW)])


def _sc_id_gather_body(utab_hbm, itab_hbm, uidx_hbm, iidx_hbm,
                       uemb_hbm, iemb_hbm, idix_v, idrows_v, sem):
    wid = lax.axis_index("s") * NC + lax.axis_index("c")
    ib = wid * PW
    pltpu.sync_copy(uidx_hbm.at[pl.ds(ib, PW)], idix_v)
    pltpu.async_copy(utab_hbm.at[idix_v], idrows_v, sem).wait()
    pltpu.sync_copy(idrows_v, uemb_hbm.at[pl.ds(ib, PW)])
    pltpu.sync_copy(iidx_hbm.at[pl.ds(ib, PW)], idix_v)
    pltpu.async_copy(itab_hbm.at[idix_v], idrows_v, sem).wait()
    pltpu.sync_copy(idrows_v, iemb_hbm.at[pl.ds(ib, PW)])


BB = 512


def _tc_body(uef, pooled, ief, qu3, qi3, g3, a3, gtab, atab,
             uW0a4, uW0g, uW0c, uW0p_pad, ub0, uW1, ub1, uW2, ub2,
             iW0a4, iW0p_pad, ib0, iW1, ib1, iW2, ib2, out):
    f32 = jnp.float32
    dot = functools.partial(jnp.dot, preferred_element_type=f32)
    lane_q = lax.broadcasted_iota(jnp.int32, (BB, 128), 1) // 32
    um = jnp.where(lane_q == qu3[0, 0, :][:, None], uef[...], 0.0)
    im = jnp.where(lane_q == qi3[0, 0, :][:, None], ief[...], 0.0)
    g = g3[0, 0, :]
    a = a3[0, 0, :]
    oh_g = (g[:, None] == lax.broadcasted_iota(jnp.int32, (BB, 3), 1)).astype(f32)
    oh_a = (a[:, None] == lax.broadcasted_iota(jnp.int32, (BB, 100), 1)).astype(f32)
    gemb = dot(oh_g, gtab[...])
    aemb = dot(oh_a, atab[...])
    u = (dot(um, uW0a4[...]) + dot(gemb, uW0g[...])
         + dot(aemb, uW0c[...]) + dot(pooled[...], uW0p_pad[...]) + ub0[...])
    u = jnp.maximum(u, 0.0)
    u = jnp.maximum(dot(u, uW1[...]) + ub1[...], 0.0)
    u = jnp.maximum(dot(u, uW2[...]) + ub2[...], 0.0)
    t = dot(im, iW0a4[...]) + dot(pooled[...], iW0p_pad[...]) + ib0[...]
    t = jnp.maximum(t, 0.0)
    t = jnp.maximum(dot(t, iW1[...]) + ib1[...], 0.0)
    t = jnp.maximum(dot(t, iW2[...]) + ib2[...], 0.0)
    num = jnp.sum(u * t, axis=-1)
    den = jnp.sqrt(jnp.sum(u * u, axis=-1)) * jnp.sqrt(jnp.sum(t * t, axis=-1))
    out[0, 0, :] = 20.0 * num / jnp.maximum(den, 1e-8)


def _full(shape):
    n = len(shape)
    return pl.BlockSpec(shape, lambda i, _n=n: (0,) * _n)


def _tc_towers(uef, pooled, ief, qu3, qi3, g3, a3, gtab, atab,
               uW0a4, uW0g, uW0c, uW0p_pad, ub0, uW1, ub1, uW2, ub2,
               iW0a4, iW0p_pad, ib0, iW1, ib1, iW2, ib2):
    nblk = B // BB
    in_specs = [
        pl.BlockSpec((BB, 128), lambda i: (i, 0)),
        pl.BlockSpec((BB, 128), lambda i: (i, 0)),
        pl.BlockSpec((BB, 128), lambda i: (i, 0)),
        pl.BlockSpec((1, 1, BB), lambda i: (i, 0, 0)),
        pl.BlockSpec((1, 1, BB), lambda i: (i, 0, 0)),
        pl.BlockSpec((1, 1, BB), lambda i: (i, 0, 0)),
        pl.BlockSpec((1, 1, BB), lambda i: (i, 0, 0)),
        _full((3, 8)), _full((100, 8)),
        _full((128, 256)), _full((8, 256)), _full((8, 256)), _full((128, 256)),
        _full((1, 256)), _full((256, 128)), _full((1, 128)),
        _full((128, 64)), _full((1, 64)),
        _full((128, 256)), _full((128, 256)), _full((1, 256)),
        _full((256, 128)), _full((1, 128)), _full((128, 64)), _full((1, 64)),
    ]
    return pl.pallas_call(
        _tc_body,
        grid=(nblk,),
        in_specs=in_specs,
        out_specs=pl.BlockSpec((1, 1, BB), lambda i: (i, 0, 0)),
        out_shape=jax.ShapeDtypeStruct((nblk, 1, BB), jnp.float32),
    )(uef, pooled, ief, qu3, qi3, g3, a3, gtab, atab,
      uW0a4, uW0g, uW0c, uW0p_pad, ub0, uW1, ub1, uW2, ub2,
      iW0a4, iW0p_pad, ib0, iW1, ib1, iW2, ib2)


def kernel(text_embed, user_id_table, gender_table, age_table, item_id_table,
           uW0, ub0, uW1, ub1, uW2, ub2,
           iW0, ib0, iW1, ib1, iW2, ib2,
           user_id, gender, age, user_hist, item_id, item_title):
    textp = jnp.pad(text_embed, ((0, 0), (0, 128 - D)))
    utabp = _repack_id(user_id_table.T, textp)
    itabp = _repack_id(item_id_table.T, textp)
    hidx = jnp.concatenate([user_hist, item_title], axis=1).reshape(-1)
    pooled = _make_sc_pool()(textp, hidx)
    uef, ief = _make_sc_id_gather()(
        utabp, itabp,
        jnp.bitwise_and(user_id[:, 0], IDQ - 1),
        jnp.bitwise_and(item_id[:, 0], IDQ - 1))
    qu3 = jnp.right_shift(user_id[:, 0], 18).reshape(B // BB, 1, BB)
    qi3 = jnp.right_shift(item_id[:, 0], 18).reshape(B // BB, 1, BB)
    g3 = gender[:, 0].reshape(B // BB, 1, BB)
    a3 = age[:, 0].reshape(B // BB, 1, BB)
    z64 = jnp.zeros((D, 256), jnp.float32)
    score = _tc_towers(
        uef, pooled, ief, qu3, qi3, g3, a3, gender_table, age_table,
        jnp.concatenate([uW0[:32]] * 4, axis=0), uW0[32:40], uW0[40:48],
        jnp.concatenate([uW0[48:112], z64], axis=0), ub0.reshape(1, 256),
        uW1, ub1.reshape(1, 128), uW2, ub2.reshape(1, 64),
        jnp.concatenate([iW0[:32]] * 4, axis=0),
        jnp.concatenate([z64, iW0[32:96]], axis=0), ib0.reshape(1, 256),
        iW1, ib1.reshape(1, 128), iW2, ib2.reshape(1, 64))
    return score.reshape(B, 1)


# speedup vs baseline: 1.2013x; 1.2013x over previous
import functools

import jax
import jax.numpy as jnp
from jax import lax
from jax.experimental import pallas as pl
from jax.experimental.pallas import tpu as pltpu
from jax.experimental.pallas import tpu_sc as plsc

B = 4096
HIST = 50
D = 64
VOCAB1 = 1000001
NC = 2
NS = 16
NW = NC * NS
PW = B // NW
IPS = 2 * HIST
CS = 4
CHUNK_ROWS = CS * IPS
N_CH = PW // CS
L = 16

IDQ = 1 << 18
ID_ROWS = IDQ
VI_COLS = 1000000
RB = 4096


def _id_repack_body(q0, q1, q2, q3, out):
    out[...] = jnp.concatenate(
        [jnp.transpose(q0[...]), jnp.transpose(q1[...]),
         jnp.transpose(q2[...]), jnp.transpose(q3[...])], axis=1)


def _repack_id(tabT):
    nblk = ID_ROWS // RB
    vblk = (VI_COLS + RB - 1) // RB - 1
    return pl.pallas_call(
        _id_repack_body,
        grid=(nblk,),
        in_specs=[
            pl.BlockSpec((32, RB),
                         lambda i, _q=q, _n=nblk, _v=vblk:
                         (0, jnp.minimum(_q * _n + i, _v)))
            for q in range(4)
        ],
        out_specs=pl.BlockSpec((RB, 128), lambda i: (i, 0)),
        out_shape=jax.ShapeDtypeStruct((ID_ROWS, 128), jnp.float32),
        compiler_params=pltpu.CompilerParams(
            dimension_semantics=("parallel",)),
    )(tabT, tabT, tabT, tabT)


@functools.cache
def _make_sc_pool():
    mesh = plsc.VectorSubcoreMesh(core_axis_name="c", subcore_axis_name="s",
                                  num_cores=NC, num_subcores=NS)
    return functools.partial(
        pl.kernel,
        out_type=jax.ShapeDtypeStruct((B, 128), jnp.float32),
        mesh=mesh,
        scratch_types=[
            pltpu.VMEM((PW * IPS,), jnp.int32),
            pltpu.VMEM((CHUNK_ROWS, 128), jnp.float32),
            pltpu.VMEM((PW, 128), jnp.float32),
            pltpu.SemaphoreType.DMA,
        ],
        compiler_params=pltpu.CompilerParams(use_tc_tiling_on_sc=True),
    )(_sc_pool_body)


@functools.cache
def _make_sc_id_gather():
    mesh = plsc.VectorSubcoreMesh(core_axis_name="c", subcore_axis_name="s",
                                  num_cores=NC, num_subcores=NS)
    return functools.partial(
        pl.kernel,
        out_type=(
            jax.ShapeDtypeStruct((B, 128), jnp.float32),
            jax.ShapeDtypeStruct((B, 128), jnp.float32),
        ),
        mesh=mesh,
        scratch_types=[
            pltpu.VMEM((PW,), jnp.int32),
            pltpu.VMEM((PW, 128), jnp.float32),
            pltpu.SemaphoreType.DMA,
        ],
        compiler_params=pltpu.CompilerParams(use_tc_tiling_on_sc=True),
    )(_sc_id_gather_body)


def _sc_pool_body(text_hbm, hidx_hbm, pooled_hbm, idx_v, rows_v, out_v, sem):
    wid = lax.axis_index("s") * NC + lax.axis_index("c")
    base = wid * PW * IPS

    pltpu.sync_copy(hidx_hbm.at[pl.ds(base, PW * IPS)], idx_v)

    @pl.loop(0, N_CH)
    def _chunk(ch):
        row0 = ch * CHUNK_ROWS
        cps = []
        for j, w in ((0, 128), (1, 128), (2, 128), (3, 16)):
            cps.append(pltpu.async_copy(
                text_hbm.at[idx_v.at[pl.ds(row0 + j * 128, w)]],
                rows_v.at[pl.ds(j * 128, w)], sem))
        for cp in cps:
            cp.wait()
        for s in range(CS):
            sp = ch * CS + s
            accs = [jnp.zeros((L,), jnp.float32) for _ in range(8)]
            for half, rbase in ((0, s * IPS), (1, s * IPS + HIST)):
                for r in range(HIST):
                    for c in range(4):
                        accs[4 * half + c] = accs[4 * half + c] + \
                            rows_v[rbase + r, pl.ds(c * L, L)]
            for c in range(8):
                out_v[sp, pl.ds(c * L, L)] = accs[c] * jnp.float32(1.0 / HIST)

    pltpu.sync_copy(out_v, pooled_hbm.at[pl.ds(wid * PW, PW)])


def _sc_id_gather_body(utab_hbm, itab_hbm, uidx_hbm, iidx_hbm,
                       uemb_hbm, iemb_hbm, idix_v, idrows_v, sem):
    wid = lax.axis_index("s") * NC + lax.axis_index("c")
    ib = wid * PW
    pltpu.sync_copy(uidx_hbm.at[pl.ds(ib, PW)], idix_v)
    pltpu.async_copy(utab_hbm.at[idix_v], idrows_v, sem).wait()
    pltpu.sync_copy(idrows_v, uemb_hbm.at[pl.ds(ib, PW)])
    pltpu.sync_copy(iidx_hbm.at[pl.ds(ib, PW)], idix_v)
    pltpu.async_copy(itab_hbm.at[idix_v], idrows_v, sem).wait()
    pltpu.sync_copy(idrows_v, iemb_hbm.at[pl.ds(ib, PW)])


BB = 512


def _tc_body(uef, pooled, ief, qu3, qi3, g3, a3, gtab, atab,
             uW0a4, uW0g, uW0c, uW0p_pad, ub0, uW1, ub1, uW2, ub2,
             iW0a4, iW0p_pad, ib0, iW1, ib1, iW2, ib2, out):
    f32 = jnp.float32
    dot = functools.partial(jnp.dot, preferred_element_type=f32)
    lane_q = lax.broadcasted_iota(jnp.int32, (BB, 128), 1) // 32
    um = jnp.where(lane_q == qu3[0, 0, :][:, None], uef[...], 0.0)
    im = jnp.where(lane_q == qi3[0, 0, :][:, None], ief[...], 0.0)
    g = g3[0, 0, :]
    a = a3[0, 0, :]
    oh_g = (g[:, None] == lax.broadcasted_iota(jnp.int32, (BB, 3), 1)).astype(f32)
    oh_a = (a[:, None] == lax.broadcasted_iota(jnp.int32, (BB, 100), 1)).astype(f32)
    gemb = dot(oh_g, gtab[...])
    aemb = dot(oh_a, atab[...])
    u = (dot(um, uW0a4[...]) + dot(gemb, uW0g[...])
         + dot(aemb, uW0c[...]) + dot(pooled[...], uW0p_pad[...]) + ub0[...])
    u = jnp.maximum(u, 0.0)
    u = jnp.maximum(dot(u, uW1[...]) + ub1[...], 0.0)
    u = jnp.maximum(dot(u, uW2[...]) + ub2[...], 0.0)
    t = dot(im, iW0a4[...]) + dot(pooled[...], iW0p_pad[...]) + ib0[...]
    t = jnp.maximum(t, 0.0)
    t = jnp.maximum(dot(t, iW1[...]) + ib1[...], 0.0)
    t = jnp.maximum(dot(t, iW2[...]) + ib2[...], 0.0)
    num = jnp.sum(u * t, axis=-1)
    den = jnp.sqrt(jnp.sum(u * u, axis=-1)) * jnp.sqrt(jnp.sum(t * t, axis=-1))
    out[0, 0, :] = 20.0 * num / jnp.maximum(den, 1e-8)


def _full(shape):
    n = len(shape)
    return pl.BlockSpec(shape, lambda i, _n=n: (0,) * _n)


def _tc_towers(uef, pooled, ief, qu3, qi3, g3, a3, gtab, atab,
               uW0a4, uW0g, uW0c, uW0p_pad, ub0, uW1, ub1, uW2, ub2,
               iW0a4, iW0p_pad, ib0, iW1, ib1, iW2, ib2):
    nblk = B // BB
    in_specs = [
        pl.BlockSpec((BB, 128), lambda i: (i, 0)),
        pl.BlockSpec((BB, 128), lambda i: (i, 0)),
        pl.BlockSpec((BB, 128), lambda i: (i, 0)),
        pl.BlockSpec((1, 1, BB), lambda i: (i, 0, 0)),
        pl.BlockSpec((1, 1, BB), lambda i: (i, 0, 0)),
        pl.BlockSpec((1, 1, BB), lambda i: (i, 0, 0)),
        pl.BlockSpec((1, 1, BB), lambda i: (i, 0, 0)),
        _full((3, 8)), _full((100, 8)),
        _full((128, 256)), _full((8, 256)), _full((8, 256)), _full((128, 256)),
        _full((1, 256)), _full((256, 128)), _full((1, 128)),
        _full((128, 64)), _full((1, 64)),
        _full((128, 256)), _full((128, 256)), _full((1, 256)),
        _full((256, 128)), _full((1, 128)), _full((128, 64)), _full((1, 64)),
    ]
    return pl.pallas_call(
        _tc_body,
        grid=(nblk,),
        in_specs=in_specs,
        out_specs=pl.BlockSpec((1, 1, BB), lambda i: (i, 0, 0)),
        out_shape=jax.ShapeDtypeStruct((nblk, 1, BB), jnp.float32),
    )(uef, pooled, ief, qu3, qi3, g3, a3, gtab, atab,
      uW0a4, uW0g, uW0c, uW0p_pad, ub0, uW1, ub1, uW2, ub2,
      iW0a4, iW0p_pad, ib0, iW1, ib1, iW2, ib2)


def kernel(text_embed, user_id_table, gender_table, age_table, item_id_table,
           uW0, ub0, uW1, ub1, uW2, ub2,
           iW0, ib0, iW1, ib1, iW2, ib2,
           user_id, gender, age, user_hist, item_id, item_title):
    textp = jnp.pad(text_embed, ((0, 0), (0, 128 - D)))
    utabp = _repack_id(user_id_table.T)
    itabp = _repack_id(item_id_table.T)
    hidx = jnp.concatenate([user_hist, item_title], axis=1).reshape(-1)
    pooled = _make_sc_pool()(textp, hidx)
    uef, ief = _make_sc_id_gather()(
        utabp, itabp,
        jnp.bitwise_and(user_id[:, 0], IDQ - 1),
        jnp.bitwise_and(item_id[:, 0], IDQ - 1))
    qu3 = jnp.right_shift(user_id[:, 0], 18).reshape(B // BB, 1, BB)
    qi3 = jnp.right_shift(item_id[:, 0], 18).reshape(B // BB, 1, BB)
    g3 = gender[:, 0].reshape(B // BB, 1, BB)
    a3 = age[:, 0].reshape(B // BB, 1, BB)
    z64 = jnp.zeros((D, 256), jnp.float32)
    score = _tc_towers(
        uef, pooled, ief, qu3, qi3, g3, a3, gender_table, age_table,
        jnp.concatenate([uW0[:32]] * 4, axis=0), uW0[32:40], uW0[40:48],
        jnp.concatenate([uW0[48:112], z64], axis=0), ub0.reshape(1, 256),
        uW1, ub1.reshape(1, 128), uW2, ub2.reshape(1, 64),
        jnp.concatenate([iW0[:32]] * 4, axis=0),
        jnp.concatenate([z64, iW0[32:96]], axis=0), ib0.reshape(1, 256),
        iW1, ib1.reshape(1, 128), iW2, ib2.reshape(1, 64))
    return score.reshape(B, 1)
